# Initial kernel scaffold; baseline (speedup 1.0000x reference)
import jax
import jax.numpy as jnp
from jax.experimental import pallas as pl


def kernel(init_logits, features, src_connect, dst_connect, src_decorate, dst_decorate, src_next, dst_next, W, b, ln_gamma, ln_beta, ew_connect, ew_decorate, ew_next):
    raise NotImplementedError("write your pallas kernel here")


# speedup vs baseline: 2.0163x; 2.0163x over previous
import functools

import jax
import jax.numpy as jnp
from jax import lax
from jax.experimental import pallas as pl
from jax.experimental.pallas import tpu as pltpu
from jax.experimental.pallas import tpu_sc as plsc

N, D, C = 10000, 128, 64
NUM_LAYERS, ALPHA = 5, 0.5
NPAD = 10240
E = 160000
NW = 32
CH = 128
EPW = 5120
EPAD = EPW * NW
NCH = EPW // CH
VCH = E // CH
RPT = NPAD // 16
BN = 400


def _transform_body(f_ref, w_ref, b_ref, g_ref, be_ref, out_ref):
    x = f_ref[...]
    t = lax.dot_general(x, w_ref[...], (((1,), (1,)), ((), ())),
                        preferred_element_type=jnp.float32)
    t = t + b_ref[...]
    mu = jnp.mean(t, axis=-1, keepdims=True)
    var = jnp.mean((t - mu) ** 2, axis=-1, keepdims=True)
    t = (t - mu) * lax.rsqrt(var + 1e-5) * g_ref[...] + be_ref[...]
    t = jnp.maximum(t, 0.0)
    nrm = jnp.sqrt(jnp.sum(t * t, axis=-1, keepdims=True))
    out_ref[...] = t / jnp.maximum(nrm, 1e-8)


def _transform(features, W, b, ln_gamma, ln_beta):
    return pl.pallas_call(
        _transform_body,
        grid=(N // BN,),
        in_specs=[
            pl.BlockSpec((BN, D), lambda i: (i, 0)),
            pl.BlockSpec((D, D), lambda i: (0, 0)),
            pl.BlockSpec((1, D), lambda i: (0, 0)),
            pl.BlockSpec((1, D), lambda i: (0, 0)),
            pl.BlockSpec((1, D), lambda i: (0, 0)),
        ],
        out_specs=pl.BlockSpec((BN, D), lambda i: (i, 0)),
        out_shape=jax.ShapeDtypeStruct((N, D), jnp.float32),
    )(features, W, b.reshape(1, D), ln_gamma.reshape(1, D),
      ln_beta.reshape(1, D))


_MESH = plsc.VectorSubcoreMesh(core_axis_name="c", subcore_axis_name="s")


@functools.partial(
    pl.kernel,
    out_type=[
        jax.ShapeDtypeStruct((3 * EPAD,), jnp.float32),
        jax.ShapeDtypeStruct((2, NPAD), jnp.float32),
    ],
    mesh=_MESH,
    compiler_params=pltpu.CompilerParams(needs_layout_passes=False, use_tc_tiling_on_sc=False),
    scratch_types=[
        pltpu.VMEM((CH,), jnp.int32),
        pltpu.VMEM((CH,), jnp.int32),
        pltpu.VMEM((CH, D), jnp.float32),
        pltpu.VMEM((CH, D), jnp.float32),
        pltpu.VMEM((256,), jnp.float32),
        pltpu.VMEM((CH,), jnp.float32),
        pltpu.VMEM((16,), jnp.float32),
        pltpu.VMEM_SHARED((NPAD,), jnp.float32),
        pltpu.SemaphoreType.DMA,
        pltpu.SemaphoreType.DMA,
    ],
)
def _weights_kernel(that_hbm, src_hbm, dst_hbm, ew_hbm, w_out, tw_out,
                    srci, dsti, srcr, dstr, dots, wv, sigv, twacc,
                    sem1, sem2):
    cid = lax.axis_index("c")
    sid = lax.axis_index("s")
    wid = sid * 2 + cid

    pltpu.sync_copy(ew_hbm, sigv)
    sig = 1.0 / (1.0 + jnp.exp(-sigv[...]))

    zeros16 = jnp.zeros((16,), jnp.float32)

    def zfill(i, _):
        wv[pl.ds(i * 16, 16)] = zeros16
        return 0

    lax.fori_loop(0, CH // 16, zfill, 0)

    def zcopy(j, _):
        pltpu.sync_copy(wv, twacc.at[pl.ds(sid * RPT + j * CH, CH)])
        return 0

    lax.fori_loop(0, RPT // CH, zcopy, 0)
    plsc.subcore_barrier()

    iota_sc = lax.iota(jnp.int32, 16) * 16

    for t in range(3):
        s_t = sig[t]

        def chunk_body(cc, _):
            base = t * EPAD + wid * EPW + cc * CH
            pltpu.sync_copy(src_hbm.at[pl.ds(base, CH)], srci)
            pltpu.sync_copy(dst_hbm.at[pl.ds(base, CH)], dsti)
            cp1 = pltpu.async_copy(that_hbm.at[srci], srcr, sem1)
            cp2 = pltpu.async_copy(that_hbm.at[dsti], dstr, sem2)
            cp1.wait()
            cp2.wait()
            validf = jnp.where(wid * NCH + cc < VCH, s_t, 0.0)

            def grp(g, _):
                for e in range(16):
                    row = g * 16 + e
                    acc = srcr[row, pl.ds(0, 16)] * dstr[row, pl.ds(0, 16)]
                    for k in range(1, 8):
                        acc = acc + (srcr[row, pl.ds(16 * k, 16)]
                                     * dstr[row, pl.ds(16 * k, 16)])
                    dots[pl.ds(e * 16, 16)] = acc
                tot = plsc.load_gather(dots, [iota_sc])
                for j in range(1, 16):
                    tot = tot + plsc.load_gather(
                        dots, [iota_sc + jnp.full((16,), j, jnp.int32)])
                wv[pl.ds(g * 16, 16)] = validf * tot
                return 0

            lax.fori_loop(0, CH // 16, grp, 0)
            pltpu.sync_copy(wv, w_out.at[pl.ds(base, CH)])
            pltpu.sync_copy(wv, twacc.at[srci], add=True)
            return 0

        lax.fori_loop(0, NCH, chunk_body, 0)

    plsc.subcore_barrier()
    pltpu.sync_copy(twacc.at[pl.ds(sid * RPT, RPT)],
                    tw_out.at[cid, pl.ds(sid * RPT, RPT)])


@functools.partial(
    pl.kernel,
    out_type=jax.ShapeDtypeStruct((2, NPAD, C), jnp.float32),
    mesh=_MESH,
    compiler_params=pltpu.CompilerParams(needs_layout_passes=False, use_tc_tiling_on_sc=False),
    scratch_types=[
        pltpu.VMEM((CH,), jnp.int32),
        pltpu.VMEM((CH,), jnp.int32),
        pltpu.VMEM((CH,), jnp.float32),
        pltpu.VMEM((CH, C), jnp.float32),
        pltpu.VMEM_SHARED((NPAD, C), jnp.float32),
        pltpu.SemaphoreType.DMA,
    ],
)
def _layer_kernel(cur_hbm, src_hbm, dst_hbm, w_hbm, p_out,
                  srci, dsti, wv, rows, acc, sem):
    cid = lax.axis_index("c")
    sid = lax.axis_index("s")
    wid = sid * 2 + cid

    zeros16 = jnp.zeros((16,), jnp.float32)

    def zrow(i, _):
        for k in range(C // 16):
            rows[i, pl.ds(16 * k, 16)] = zeros16
        return 0

    lax.fori_loop(0, CH, zrow, 0)

    def zcopy(j, _):
        pltpu.sync_copy(rows, acc.at[pl.ds(sid * RPT + j * CH, CH)])
        return 0

    lax.fori_loop(0, RPT // CH, zcopy, 0)
    plsc.subcore_barrier()

    for t in range(3):

        def chunk_body(cc, _):
            base = t * EPAD + wid * EPW + cc * CH
            pltpu.sync_copy(src_hbm.at[pl.ds(base, CH)], srci)
            pltpu.sync_copy(dst_hbm.at[pl.ds(base, CH)], dsti)
            pltpu.sync_copy(w_hbm.at[pl.ds(base, CH)], wv)
            pltpu.async_copy(cur_hbm.at[dsti], rows, sem).wait()

            def grp(g, _):
                for e in range(8):
                    row = g * 8 + e
                    wb = plsc.load_gather(wv, [jnp.full((16,), row, jnp.int32)])
                    for k in range(C // 16):
                        rows[row, pl.ds(16 * k, 16)] = (
                            rows[row, pl.ds(16 * k, 16)] * wb)
                return 0

            lax.fori_loop(0, CH // 8, grp, 0)
            pltpu.sync_copy(rows, acc.at[srci], add=True)
            return 0

        lax.fori_loop(0, NCH, chunk_body, 0)

    plsc.subcore_barrier()
    pltpu.sync_copy(acc.at[pl.ds(sid * RPT, RPT)],
                    p_out.at[cid, pl.ds(sid * RPT, RPT)])


def _epilogue_body(p_ref, tw_ref, init_ref, out_ref):
    psum = p_ref[0] + p_ref[1]
    tw = tw_ref[0] + tw_ref[1]
    scale = jnp.where(tw > 0, 1.0 / tw, 1.0)
    out_ref[...] = ALPHA * psum * scale + (1.0 - ALPHA) * init_ref[...]


def _epilogue(p, twp, init_logits):
    return pl.pallas_call(
        _epilogue_body,
        grid=(N // BN,),
        in_specs=[
            pl.BlockSpec((2, BN, C), lambda i: (0, i, 0)),
            pl.BlockSpec((2, BN, 1), lambda i: (0, i, 0)),
            pl.BlockSpec((BN, C), lambda i: (i, 0)),
        ],
        out_specs=pl.BlockSpec((BN, C), lambda i: (i, 0)),
        out_shape=jax.ShapeDtypeStruct((N, C), jnp.float32),
    )(p, twp, init_logits)


def kernel(init_logits, features, src_connect, dst_connect, src_decorate,
           dst_decorate, src_next, dst_next, W, b, ln_gamma, ln_beta,
           ew_connect, ew_decorate, ew_next):
    pad = jnp.zeros((EPAD - E,), jnp.int32)
    src = jnp.concatenate([src_connect, pad, src_decorate, pad, src_next, pad])
    dst = jnp.concatenate([dst_connect, pad, dst_decorate, pad, dst_next, pad])
    ew = jnp.concatenate([ew_connect, ew_decorate, ew_next,
                          jnp.zeros((13,), jnp.float32)])

    that = _transform(features, W, b, ln_gamma, ln_beta)
    w_e, twp = _weights_kernel(that, src, dst, ew)
    twp3 = twp.reshape(2, NPAD, 1)

    cur = init_logits
    for _ in range(NUM_LAYERS):
        p = _layer_kernel(cur, src, dst, w_e)
        cur = _epilogue(p, twp3, init_logits)
    return cur


# speedup vs baseline: 2.8886x; 1.4326x over previous
import functools

import jax
import jax.numpy as jnp
from jax import lax
from jax.experimental import pallas as pl
from jax.experimental.pallas import tpu as pltpu
from jax.experimental.pallas import tpu_sc as plsc

N, D, C = 10000, 128, 64
NUM_LAYERS, ALPHA = 5, 0.5
NPAD = 10240
E = 160000
NW = 32
CH = 128
EPW = 5120
EPAD = EPW * NW
NCH = EPW // CH
VCH = E // CH
RPT = NPAD // 16
BN = 400


def _transform_body(f_ref, w_ref, b_ref, g_ref, be_ref, out_ref):
    x = f_ref[...]
    t = lax.dot_general(x, w_ref[...], (((1,), (1,)), ((), ())),
                        preferred_element_type=jnp.float32)
    t = t + b_ref[...]
    mu = jnp.mean(t, axis=-1, keepdims=True)
    var = jnp.mean((t - mu) ** 2, axis=-1, keepdims=True)
    t = (t - mu) * lax.rsqrt(var + 1e-5) * g_ref[...] + be_ref[...]
    t = jnp.maximum(t, 0.0)
    nrm = jnp.sqrt(jnp.sum(t * t, axis=-1, keepdims=True))
    out_ref[...] = t / jnp.maximum(nrm, 1e-8)


def _transform(features, W, b, ln_gamma, ln_beta):
    return pl.pallas_call(
        _transform_body,
        grid=(N // BN,),
        in_specs=[
            pl.BlockSpec((BN, D), lambda i: (i, 0)),
            pl.BlockSpec((D, D), lambda i: (0, 0)),
            pl.BlockSpec((1, D), lambda i: (0, 0)),
            pl.BlockSpec((1, D), lambda i: (0, 0)),
            pl.BlockSpec((1, D), lambda i: (0, 0)),
        ],
        out_specs=pl.BlockSpec((BN, D), lambda i: (i, 0)),
        out_shape=jax.ShapeDtypeStruct((N, D), jnp.float32),
    )(features, W, b.reshape(1, D), ln_gamma.reshape(1, D),
      ln_beta.reshape(1, D))


_MESH = plsc.VectorSubcoreMesh(core_axis_name="c", subcore_axis_name="s")


@functools.partial(
    pl.kernel,
    out_type=[
        jax.ShapeDtypeStruct((3 * EPAD,), jnp.float32),
        jax.ShapeDtypeStruct((2, NPAD), jnp.float32),
    ],
    mesh=_MESH,
    compiler_params=pltpu.CompilerParams(needs_layout_passes=False, use_tc_tiling_on_sc=False),
    scratch_types=[
        pltpu.VMEM((CH,), jnp.int32),
        pltpu.VMEM((CH,), jnp.int32),
        pltpu.VMEM((CH, D), jnp.float32),
        pltpu.VMEM((CH, D), jnp.float32),
        pltpu.VMEM((256,), jnp.float32),
        pltpu.VMEM((CH,), jnp.float32),
        pltpu.VMEM((16,), jnp.float32),
        pltpu.VMEM_SHARED((NPAD,), jnp.float32),
        pltpu.SemaphoreType.DMA,
        pltpu.SemaphoreType.DMA,
    ],
)
def _weights_kernel(that_hbm, src_hbm, dst_hbm, ew_hbm, w_out, tw_out,
                    srci, dsti, srcr, dstr, dots, wv, sigv, twacc,
                    sem1, sem2):
    cid = lax.axis_index("c")
    sid = lax.axis_index("s")
    wid = sid * 2 + cid

    pltpu.sync_copy(ew_hbm, sigv)
    sig = 1.0 / (1.0 + jnp.exp(-sigv[...]))

    zeros16 = jnp.zeros((16,), jnp.float32)

    def zfill(i, _):
        wv[pl.ds(i * 16, 16)] = zeros16
        return 0

    lax.fori_loop(0, CH // 16, zfill, 0)

    def zcopy(j, _):
        pltpu.sync_copy(wv, twacc.at[pl.ds(sid * RPT + j * CH, CH)])
        return 0

    lax.fori_loop(0, RPT // CH, zcopy, 0)
    plsc.subcore_barrier()

    iota_sc = lax.iota(jnp.int32, 16) * 16

    for t in range(3):
        s_t = sig[t]

        def chunk_body(cc, _):
            base = t * EPAD + wid * EPW + cc * CH
            pltpu.sync_copy(src_hbm.at[pl.ds(base, CH)], srci)
            pltpu.sync_copy(dst_hbm.at[pl.ds(base, CH)], dsti)
            cp1 = pltpu.async_copy(that_hbm.at[srci], srcr, sem1)
            cp2 = pltpu.async_copy(that_hbm.at[dsti], dstr, sem2)
            cp1.wait()
            cp2.wait()
            validf = jnp.where(wid * NCH + cc < VCH, s_t, 0.0)

            def grp(g, _):
                for e in range(16):
                    row = g * 16 + e
                    acc = srcr[row, pl.ds(0, 16)] * dstr[row, pl.ds(0, 16)]
                    for k in range(1, 8):
                        acc = acc + (srcr[row, pl.ds(16 * k, 16)]
                                     * dstr[row, pl.ds(16 * k, 16)])
                    dots[pl.ds(e * 16, 16)] = acc
                tot = plsc.load_gather(dots, [iota_sc])
                for j in range(1, 16):
                    tot = tot + plsc.load_gather(
                        dots, [iota_sc + jnp.full((16,), j, jnp.int32)])
                wv[pl.ds(g * 16, 16)] = validf * tot
                return 0

            lax.fori_loop(0, CH // 16, grp, 0)
            pltpu.sync_copy(wv, w_out.at[pl.ds(base, CH)])
            pltpu.sync_copy(wv, twacc.at[srci], add=True)
            return 0

        lax.fori_loop(0, NCH, chunk_body, 0)

    plsc.subcore_barrier()
    pltpu.sync_copy(twacc.at[pl.ds(sid * RPT, RPT)],
                    tw_out.at[cid, pl.ds(sid * RPT, RPT)])


@functools.partial(
    pl.kernel,
    out_type=jax.ShapeDtypeStruct((2, NPAD, C), jnp.float32),
    mesh=_MESH,
    compiler_params=pltpu.CompilerParams(needs_layout_passes=False,
                                         use_tc_tiling_on_sc=False),
    scratch_types=[
        pltpu.VMEM((16, CH), jnp.int32),
        pltpu.VMEM((16, CH), jnp.int32),
        pltpu.VMEM((4 * 512,), jnp.float32),
        pltpu.VMEM((1024, C), jnp.float32),
        pltpu.VMEM_SHARED((NPAD, C), jnp.float32),
        pltpu.SemaphoreType.DMA((4,)),
        pltpu.SemaphoreType.DMA((2,)),
        pltpu.SemaphoreType.DMA,
    ],
)
def _layer_kernel(cur_hbm, src_hbm, dst_hbm, w_hbm, p_out,
                  srci, dsti, wv, rows, acc, semi, semg, sems):
    cid = lax.axis_index("c")
    sid = lax.axis_index("s")
    wid = sid * 2 + cid
    NT = 3 * EPW // 512

    def ebase(c):
        t = lax.div(c, NT // 3)
        return t * EPAD + wid * EPW + (c - t * (NT // 3)) * 512

    def idx_descs(c):
        base = ebase(c)
        slot = lax.rem(c, 4)
        ds = []
        for j in range(4):
            ds.append((src_hbm.at[pl.ds(base + j * CH, CH)],
                       srci.at[slot * 4 + j], semi.at[slot]))
            ds.append((dst_hbm.at[pl.ds(base + j * CH, CH)],
                       dsti.at[slot * 4 + j], semi.at[slot]))
        ds.append((w_hbm.at[pl.ds(base, 512)],
                   wv.at[pl.ds(slot * 512, 512)], semi.at[slot]))
        return ds

    def gat_descs(c):
        slot = lax.rem(c, 4)
        rbase = lax.rem(c, 2) * 512
        return [(cur_hbm.at[dsti.at[slot * 4 + j]],
                 rows.at[pl.ds(rbase + j * CH, CH)], semg.at[lax.rem(c, 2)])
                for j in range(4)]

    def sct_descs(c):
        slot = lax.rem(c, 4)
        rbase = lax.rem(c, 2) * 512
        return [(rows.at[pl.ds(rbase + j * CH, CH)],
                 acc.at[srci.at[slot * 4 + j]], sems)
                for j in range(4)]

    def fire(ds, add=False):
        for s, d, sem in ds:
            pltpu.async_copy(s, d, sem, add=add)

    def drain(ds):
        for s, d, sem in ds:
            pltpu.make_async_copy(s, d, sem).wait()

    zeros16 = jnp.zeros((16,), jnp.float32)

    fire(idx_descs(0))
    fire(idx_descs(1))

    def zrow(i, _):
        for k in range(C // 16):
            rows[i, pl.ds(16 * k, 16)] = zeros16
        return 0

    lax.fori_loop(0, CH, zrow, 0)

    def zcopy(j, _):
        pltpu.sync_copy(rows.at[pl.ds(0, CH)],
                        acc.at[pl.ds(sid * RPT + j * CH, CH)])
        return 0

    lax.fori_loop(0, RPT // CH, zcopy, 0)
    plsc.subcore_barrier()

    drain(idx_descs(0))
    fire(gat_descs(0))

    def chunk_body(c, _):

        @pl.when(c >= 1)
        def _():
            drain(sct_descs(c - 1))

        @pl.when(c + 2 < NT)
        def _():
            fire(idx_descs(c + 2))

        @pl.when(c + 1 < NT)
        def _():
            drain(idx_descs(c + 1))
            fire(gat_descs(c + 1))

        drain(gat_descs(c))
        rbase = lax.rem(c, 2) * 512
        wbase = lax.rem(c, 4) * 512

        def grp(g, _):
            for e in range(8):
                k = g * 8 + e
                wb = plsc.load_gather(
                    wv, [jnp.full((16,), wbase + k, jnp.int32)])
                row = rbase + k
                for q in range(C // 16):
                    rows[row, pl.ds(16 * q, 16)] = (
                        rows[row, pl.ds(16 * q, 16)] * wb)
            return 0

        lax.fori_loop(0, 512 // 8, grp, 0)
        fire(sct_descs(c), add=True)
        return 0

    lax.fori_loop(0, NT, chunk_body, 0)
    drain(sct_descs(NT - 1))

    plsc.subcore_barrier()
    pltpu.sync_copy(acc.at[pl.ds(sid * RPT, RPT)],
                    p_out.at[cid, pl.ds(sid * RPT, RPT)])


def _epilogue_body(p_ref, tw_ref, init_ref, out_ref):
    psum = p_ref[0] + p_ref[1]
    tw = tw_ref[0] + tw_ref[1]
    scale = jnp.where(tw > 0, 1.0 / tw, 1.0)
    out_ref[...] = ALPHA * psum * scale + (1.0 - ALPHA) * init_ref[...]


def _epilogue(p, twp, init_logits):
    return pl.pallas_call(
        _epilogue_body,
        grid=(N // BN,),
        in_specs=[
            pl.BlockSpec((2, BN, C), lambda i: (0, i, 0)),
            pl.BlockSpec((2, BN, 1), lambda i: (0, i, 0)),
            pl.BlockSpec((BN, C), lambda i: (i, 0)),
        ],
        out_specs=pl.BlockSpec((BN, C), lambda i: (i, 0)),
        out_shape=jax.ShapeDtypeStruct((N, C), jnp.float32),
    )(p, twp, init_logits)


def kernel(init_logits, features, src_connect, dst_connect, src_decorate,
           dst_decorate, src_next, dst_next, W, b, ln_gamma, ln_beta,
           ew_connect, ew_decorate, ew_next):
    pad = jnp.zeros((EPAD - E,), jnp.int32)
    src = jnp.concatenate([src_connect, pad, src_decorate, pad, src_next, pad])
    dst = jnp.concatenate([dst_connect, pad, dst_decorate, pad, dst_next, pad])
    ew = jnp.concatenate([ew_connect, ew_decorate, ew_next,
                          jnp.zeros((13,), jnp.float32)])

    that = _transform(features, W, b, ln_gamma, ln_beta)
    w_e, twp = _weights_kernel(that, src, dst, ew)
    twp3 = twp.reshape(2, NPAD, 1)

    cur = init_logits
    for _ in range(NUM_LAYERS):
        p = _layer_kernel(cur, src, dst, w_e)
        cur = _epilogue(p, twp3, init_logits)
    return cur


# speedup vs baseline: 2.9968x; 1.0375x over previous
import functools

import jax
import jax.numpy as jnp
from jax import lax
from jax.experimental import pallas as pl
from jax.experimental.pallas import tpu as pltpu
from jax.experimental.pallas import tpu_sc as plsc

N, D, C = 10000, 128, 64
NUM_LAYERS, ALPHA = 5, 0.5
NPAD = 10240
E = 160000
NW = 32
CH = 128
EPW = 5120
EPAD = EPW * NW
NCH = EPW // CH
VCH = E // CH
RPT = NPAD // 16
BN = 400


def _transform_body(f_ref, w_ref, b_ref, g_ref, be_ref, out_ref):
    x = f_ref[...]
    t = lax.dot_general(x, w_ref[...], (((1,), (1,)), ((), ())),
                        preferred_element_type=jnp.float32)
    t = t + b_ref[...]
    mu = jnp.mean(t, axis=-1, keepdims=True)
    var = jnp.mean((t - mu) ** 2, axis=-1, keepdims=True)
    t = (t - mu) * lax.rsqrt(var + 1e-5) * g_ref[...] + be_ref[...]
    t = jnp.maximum(t, 0.0)
    nrm = jnp.sqrt(jnp.sum(t * t, axis=-1, keepdims=True))
    out_ref[...] = t / jnp.maximum(nrm, 1e-8)


def _transform(features, W, b, ln_gamma, ln_beta):
    return pl.pallas_call(
        _transform_body,
        grid=(N // BN,),
        in_specs=[
            pl.BlockSpec((BN, D), lambda i: (i, 0)),
            pl.BlockSpec((D, D), lambda i: (0, 0)),
            pl.BlockSpec((1, D), lambda i: (0, 0)),
            pl.BlockSpec((1, D), lambda i: (0, 0)),
            pl.BlockSpec((1, D), lambda i: (0, 0)),
        ],
        out_specs=pl.BlockSpec((BN, D), lambda i: (i, 0)),
        out_shape=jax.ShapeDtypeStruct((N, D), jnp.float32),
    )(features, W, b.reshape(1, D), ln_gamma.reshape(1, D),
      ln_beta.reshape(1, D))


_MESH = plsc.VectorSubcoreMesh(core_axis_name="c", subcore_axis_name="s")


@functools.partial(
    pl.kernel,
    out_type=[
        jax.ShapeDtypeStruct((3 * EPAD,), jnp.float32),
        jax.ShapeDtypeStruct((2, NPAD, 16), jnp.float32),
    ],
    mesh=_MESH,
    compiler_params=pltpu.CompilerParams(needs_layout_passes=False,
                                         use_tc_tiling_on_sc=False),
    scratch_types=[
        pltpu.VMEM((120, CH), jnp.int32),
        pltpu.VMEM((120, CH), jnp.int32),
        pltpu.VMEM((256, D), jnp.float32),
        pltpu.VMEM((256, D), jnp.float32),
        pltpu.VMEM((256,), jnp.float32),
        pltpu.VMEM((120 * CH,), jnp.float32),
        pltpu.VMEM((256, 16), jnp.float32),
        pltpu.VMEM((16,), jnp.float32),
        pltpu.VMEM_SHARED((NPAD, 16), jnp.float32),
        pltpu.SemaphoreType.DMA,
        pltpu.SemaphoreType.DMA((2,)),
        pltpu.SemaphoreType.DMA((2,)),
    ],
)
def _weights_kernel(that_hbm, src2_hbm, dst2_hbm, ew_hbm, w_out, tw_out,
                    srci, dsti, srcr, dstr, dots, wall, w16, sigv, twacc,
                    semi, semg, sema):
    cid = lax.axis_index("c")
    sid = lax.axis_index("s")
    wid = sid * 2 + cid
    NT = 120
    NTT = NT // 3

    idescs = [(src2_hbm.at[pl.ds(t * 1280 + wid * NTT, NTT)],
               srci.at[pl.ds(t * NTT, NTT)], semi) for t in range(3)]
    idescs += [(dst2_hbm.at[pl.ds(t * 1280 + wid * NTT, NTT)],
                dsti.at[pl.ds(t * NTT, NTT)], semi) for t in range(3)]
    for s, d, sem in idescs:
        pltpu.async_copy(s, d, sem)

    pltpu.sync_copy(ew_hbm, sigv)
    sig = 1.0 / (1.0 + jnp.exp(-sigv[...]))
    s0, s1, s2 = sig[0], sig[1], sig[2]

    zeros16 = jnp.zeros((16,), jnp.float32)

    def zfill(i, _):
        w16[i, :] = zeros16
        return 0

    lax.fori_loop(0, CH, zfill, 0)

    def zcopy(j, _):
        pltpu.sync_copy(w16.at[pl.ds(0, CH)],
                        twacc.at[pl.ds(sid * RPT + j * CH, CH)])
        return 0

    lax.fori_loop(0, RPT // CH, zcopy, 0)
    plsc.subcore_barrier()

    for item in idescs:
        pltpu.make_async_copy(*item).wait()

    def gat_descs(c):
        rbase = lax.rem(c, 2) * CH
        return [(that_hbm.at[srci.at[c]],
                 srcr.at[pl.ds(rbase, CH)], semg.at[lax.rem(c, 2)]),
                (that_hbm.at[dsti.at[c]],
                 dstr.at[pl.ds(rbase, CH)], semg.at[lax.rem(c, 2)])]

    def add_descs(c):
        rbase = lax.rem(c, 2) * CH
        return [(w16.at[pl.ds(rbase, CH)],
                 twacc.at[srci.at[c]], sema.at[lax.rem(c, 2)])]

    def fire(ds, add=False):
        for s, d, sem in ds:
            pltpu.async_copy(s, d, sem, add=add)

    def drain(ds):
        for item in ds:
            pltpu.make_async_copy(*item).wait()

    fire(gat_descs(0))

    iota_sc = lax.iota(jnp.int32, 16) * 16

    def chunk_body(c, _):

        @pl.when(c + 1 < NT)
        def _():
            fire(gat_descs(c + 1))

        drain(gat_descs(c))

        @pl.when(c >= 1)
        def _():
            drain(add_descs(c - 1))

        rbase = lax.rem(c, 2) * CH
        t = lax.div(c, NTT)
        s_t = jnp.where(t == 0, s0, jnp.where(t == 1, s1, s2))
        validf = jnp.where(wid * NTT + (c - t * NTT) < VCH, s_t, 0.0)

        def grp(g, _):
            for e in range(16):
                row = rbase + g * 16 + e
                acc = srcr[row, pl.ds(0, 16)] * dstr[row, pl.ds(0, 16)]
                for k in range(1, 8):
                    acc = acc + (srcr[row, pl.ds(16 * k, 16)]
                                 * dstr[row, pl.ds(16 * k, 16)])
                dots[pl.ds(e * 16, 16)] = acc
            tot = plsc.load_gather(dots, [iota_sc])
            for j in range(1, 16):
                tot = tot + plsc.load_gather(
                    dots, [iota_sc + jnp.full((16,), j, jnp.int32)])
            wbase = c * CH + g * 16
            wall[pl.ds(wbase, 16)] = validf * tot
            for e in range(16):
                w16[rbase + g * 16 + e, :] = plsc.load_gather(
                    wall, [jnp.full((16,), wbase + e, jnp.int32)])
            return 0

        lax.fori_loop(0, CH // 16, grp, 0)
        fire(add_descs(c), add=True)
        return 0

    lax.fori_loop(0, NT, chunk_body, 0)
    drain(add_descs(NT - 1))

    for t in range(3):
        pltpu.sync_copy(wall.at[pl.ds(t * EPW, EPW)],
                        w_out.at[pl.ds(t * EPAD + wid * EPW, EPW)])

    plsc.subcore_barrier()
    pltpu.sync_copy(twacc.at[pl.ds(sid * RPT, RPT)],
                    tw_out.at[cid, pl.ds(sid * RPT, RPT)])


@functools.partial(
    pl.kernel,
    out_type=jax.ShapeDtypeStruct((2, NPAD, C), jnp.float32),
    mesh=_MESH,
    compiler_params=pltpu.CompilerParams(needs_layout_passes=False,
                                         use_tc_tiling_on_sc=False),
    scratch_types=[
        pltpu.VMEM((16, CH), jnp.int32),
        pltpu.VMEM((16, CH), jnp.int32),
        pltpu.VMEM((4 * 512,), jnp.float32),
        pltpu.VMEM((1024, C), jnp.float32),
        pltpu.VMEM_SHARED((NPAD, C), jnp.float32),
        pltpu.SemaphoreType.DMA((4,)),
        pltpu.SemaphoreType.DMA((2,)),
        pltpu.SemaphoreType.DMA,
    ],
)
def _layer_kernel(cur_hbm, src_hbm, dst_hbm, w_hbm, p_out,
                  srci, dsti, wv, rows, acc, semi, semg, sems):
    cid = lax.axis_index("c")
    sid = lax.axis_index("s")
    wid = sid * 2 + cid
    NT = 3 * EPW // 512

    def ebase(c):
        t = lax.div(c, NT // 3)
        return t * EPAD + wid * EPW + (c - t * (NT // 3)) * 512

    def idx_descs(c):
        base = ebase(c)
        slot = lax.rem(c, 4)
        ds = []
        for j in range(4):
            ds.append((src_hbm.at[pl.ds(base + j * CH, CH)],
                       srci.at[slot * 4 + j], semi.at[slot]))
            ds.append((dst_hbm.at[pl.ds(base + j * CH, CH)],
                       dsti.at[slot * 4 + j], semi.at[slot]))
        ds.append((w_hbm.at[pl.ds(base, 512)],
                   wv.at[pl.ds(slot * 512, 512)], semi.at[slot]))
        return ds

    def gat_descs(c):
        slot = lax.rem(c, 4)
        rbase = lax.rem(c, 2) * 512
        return [(cur_hbm.at[dsti.at[slot * 4 + j]],
                 rows.at[pl.ds(rbase + j * CH, CH)], semg.at[lax.rem(c, 2)])
                for j in range(4)]

    def sct_descs(c):
        slot = lax.rem(c, 4)
        rbase = lax.rem(c, 2) * 512
        return [(rows.at[pl.ds(rbase + j * CH, CH)],
                 acc.at[srci.at[slot * 4 + j]], sems)
                for j in range(4)]

    def fire(ds, add=False):
        for s, d, sem in ds:
            pltpu.async_copy(s, d, sem, add=add)

    def drain(ds):
        for s, d, sem in ds:
            pltpu.make_async_copy(s, d, sem).wait()

    zeros16 = jnp.zeros((16,), jnp.float32)

    fire(idx_descs(0))
    fire(idx_descs(1))

    def zrow(i, _):
        for k in range(C // 16):
            rows[i, pl.ds(16 * k, 16)] = zeros16
        return 0

    lax.fori_loop(0, CH, zrow, 0)

    def zcopy(j, _):
        pltpu.sync_copy(rows.at[pl.ds(0, CH)],
                        acc.at[pl.ds(sid * RPT + j * CH, CH)])
        return 0

    lax.fori_loop(0, RPT // CH, zcopy, 0)
    plsc.subcore_barrier()

    drain(idx_descs(0))
    fire(gat_descs(0))

    def chunk_body(c, _):

        @pl.when(c >= 1)
        def _():
            drain(sct_descs(c - 1))

        @pl.when(c + 2 < NT)
        def _():
            fire(idx_descs(c + 2))

        @pl.when(c + 1 < NT)
        def _():
            drain(idx_descs(c + 1))
            fire(gat_descs(c + 1))

        drain(gat_descs(c))
        rbase = lax.rem(c, 2) * 512
        wbase = lax.rem(c, 4) * 512

        def grp(g, _):
            for e in range(8):
                k = g * 8 + e
                wb = plsc.load_gather(
                    wv, [jnp.full((16,), wbase + k, jnp.int32)])
                row = rbase + k
                for q in range(C // 16):
                    rows[row, pl.ds(16 * q, 16)] = (
                        rows[row, pl.ds(16 * q, 16)] * wb)
            return 0

        lax.fori_loop(0, 512 // 8, grp, 0)
        fire(sct_descs(c), add=True)
        return 0

    lax.fori_loop(0, NT, chunk_body, 0)
    drain(sct_descs(NT - 1))

    plsc.subcore_barrier()
    pltpu.sync_copy(acc.at[pl.ds(sid * RPT, RPT)],
                    p_out.at[cid, pl.ds(sid * RPT, RPT)])


def _epilogue_body(p_ref, tw_ref, init_ref, out_ref):
    psum = p_ref[0] + p_ref[1]
    tw = tw_ref[0, :, 0:1] + tw_ref[1, :, 0:1]
    scale = jnp.where(tw > 0, 1.0 / tw, 1.0)
    out_ref[...] = ALPHA * psum * scale + (1.0 - ALPHA) * init_ref[...]


def _epilogue(p, twp, init_logits):
    return pl.pallas_call(
        _epilogue_body,
        grid=(N // BN,),
        in_specs=[
            pl.BlockSpec((2, BN, C), lambda i: (0, i, 0)),
            pl.BlockSpec((2, BN, 16), lambda i: (0, i, 0)),
            pl.BlockSpec((BN, C), lambda i: (i, 0)),
        ],
        out_specs=pl.BlockSpec((BN, C), lambda i: (i, 0)),
        out_shape=jax.ShapeDtypeStruct((N, C), jnp.float32),
    )(p, twp, init_logits)


def kernel(init_logits, features, src_connect, dst_connect, src_decorate,
           dst_decorate, src_next, dst_next, W, b, ln_gamma, ln_beta,
           ew_connect, ew_decorate, ew_next):
    pad = jnp.zeros((EPAD - E,), jnp.int32)
    src = jnp.concatenate([src_connect, pad, src_decorate, pad, src_next, pad])
    dst = jnp.concatenate([dst_connect, pad, dst_decorate, pad, dst_next, pad])
    ew = jnp.concatenate([ew_connect, ew_decorate, ew_next,
                          jnp.zeros((13,), jnp.float32)])

    that = _transform(features, W, b, ln_gamma, ln_beta)
    w_e, twp3 = _weights_kernel(that, src.reshape(3840, CH),
                                dst.reshape(3840, CH), ew)

    cur = init_logits
    for _ in range(NUM_LAYERS):
        p = _layer_kernel(cur, src, dst, w_e)
        cur = _epilogue(p, twp3, init_logits)
    return cur


# speedup vs baseline: 3.7134x; 1.2391x over previous
import functools

import jax
import jax.numpy as jnp
from jax import lax
from jax.experimental import pallas as pl
from jax.experimental.pallas import tpu as pltpu
from jax.experimental.pallas import tpu_sc as plsc

N, D, C = 10000, 128, 64
NUM_LAYERS, ALPHA = 5, 0.5
NPAD = 10240
E = 160000
NW = 32
CH = 128
EPW = 5120
EPAD = EPW * NW
NCH = EPW // CH
VCH = E // CH
RPT = NPAD // 16
BN = 400


def _transform_body(f_ref, w_ref, b_ref, g_ref, be_ref, out_ref):
    x = f_ref[...]
    t = lax.dot_general(x, w_ref[...], (((1,), (1,)), ((), ())),
                        preferred_element_type=jnp.float32)
    t = t + b_ref[...]
    mu = jnp.mean(t, axis=-1, keepdims=True)
    var = jnp.mean((t - mu) ** 2, axis=-1, keepdims=True)
    t = (t - mu) * lax.rsqrt(var + 1e-5) * g_ref[...] + be_ref[...]
    t = jnp.maximum(t, 0.0)
    nrm = jnp.sqrt(jnp.sum(t * t, axis=-1, keepdims=True))
    out_ref[...] = t / jnp.maximum(nrm, 1e-8)


def _transform(features, W, b, ln_gamma, ln_beta):
    return pl.pallas_call(
        _transform_body,
        grid=(N // BN,),
        in_specs=[
            pl.BlockSpec((BN, D), lambda i: (i, 0)),
            pl.BlockSpec((D, D), lambda i: (0, 0)),
            pl.BlockSpec((1, D), lambda i: (0, 0)),
            pl.BlockSpec((1, D), lambda i: (0, 0)),
            pl.BlockSpec((1, D), lambda i: (0, 0)),
        ],
        out_specs=pl.BlockSpec((BN, D), lambda i: (i, 0)),
        out_shape=jax.ShapeDtypeStruct((N, D), jnp.float32),
    )(features, W, b.reshape(1, D), ln_gamma.reshape(1, D),
      ln_beta.reshape(1, D))


_MESH = plsc.VectorSubcoreMesh(core_axis_name="c", subcore_axis_name="s")


@functools.partial(
    pl.kernel,
    out_type=[
        jax.ShapeDtypeStruct((3 * EPAD,), jnp.float32),
        jax.ShapeDtypeStruct((2, NPAD, 16), jnp.float32),
    ],
    mesh=_MESH,
    compiler_params=pltpu.CompilerParams(needs_layout_passes=False,
                                         use_tc_tiling_on_sc=False),
    scratch_types=[
        pltpu.VMEM((120, CH), jnp.int32),
        pltpu.VMEM((120, CH), jnp.int32),
        pltpu.VMEM((512, D // 2), jnp.float32),
        pltpu.VMEM((512, D // 2), jnp.float32),
        pltpu.VMEM((256,), jnp.float32),
        pltpu.VMEM((120 * CH,), jnp.float32),
        pltpu.VMEM((256, 16), jnp.float32),
        pltpu.VMEM((16,), jnp.float32),
        pltpu.VMEM_SHARED((NPAD, 16), jnp.float32),
        pltpu.SemaphoreType.DMA,
        pltpu.SemaphoreType.DMA((4,)),
        pltpu.SemaphoreType.DMA((2,)),
    ],
)
def _weights_kernel(that_hbm, src2_hbm, dst2_hbm, ew_hbm, w_out, tw_out,
                    srci, dsti, srcr, dstr, dots, wall, w16, sigv, twacc,
                    semi, semg, sema):
    cid = lax.axis_index("c")
    sid = lax.axis_index("s")
    wid = sid * 2 + cid
    NT = 120
    NTT = NT // 3

    idescs = [(src2_hbm.at[pl.ds(t * 1280 + wid * NTT, NTT)],
               srci.at[pl.ds(t * NTT, NTT)], semi) for t in range(3)]
    idescs += [(dst2_hbm.at[pl.ds(t * 1280 + wid * NTT, NTT)],
                dsti.at[pl.ds(t * NTT, NTT)], semi) for t in range(3)]
    for s, d, sem in idescs:
        pltpu.async_copy(s, d, sem)

    pltpu.sync_copy(ew_hbm, sigv)
    sig = 1.0 / (1.0 + jnp.exp(-sigv[...]))
    s0, s1, s2 = sig[0], sig[1], sig[2]

    zeros16 = jnp.zeros((16,), jnp.float32)

    def zfill(i, _):
        w16[i, :] = zeros16
        return 0

    lax.fori_loop(0, CH, zfill, 0)

    def zcopy(j, _):
        pltpu.sync_copy(w16.at[pl.ds(0, CH)],
                        twacc.at[pl.ds(sid * RPT + j * CH, CH)])
        return 0

    lax.fori_loop(0, RPT // CH, zcopy, 0)
    plsc.subcore_barrier()

    for item in idescs:
        pltpu.make_async_copy(*item).wait()

    def gat_descs(c):
        rbase = lax.rem(c, 4) * CH
        return [(that_hbm.at[srci.at[c]],
                 srcr.at[pl.ds(rbase, CH)], semg.at[lax.rem(c, 4)]),
                (that_hbm.at[dsti.at[c]],
                 dstr.at[pl.ds(rbase, CH)], semg.at[lax.rem(c, 4)])]

    def add_descs(c):
        rbase = lax.rem(c, 2) * CH
        return [(w16.at[pl.ds(rbase, CH)],
                 twacc.at[srci.at[c]], sema.at[lax.rem(c, 2)])]

    def fire(ds, add=False):
        for s, d, sem in ds:
            pltpu.async_copy(s, d, sem, add=add)

    def drain(ds):
        for item in ds:
            pltpu.make_async_copy(*item).wait()

    fire(gat_descs(0))
    fire(gat_descs(1))
    fire(gat_descs(2))

    iota_sc = lax.iota(jnp.int32, 16) * 16

    def chunk_body(c, _):

        @pl.when(c + 3 < NT)
        def _():
            fire(gat_descs(c + 3))

        drain(gat_descs(c))

        @pl.when(c >= 1)
        def _():
            drain(add_descs(c - 1))

        rbase = lax.rem(c, 4) * CH
        t = lax.div(c, NTT)
        s_t = jnp.where(t == 0, s0, jnp.where(t == 1, s1, s2))
        validf = jnp.where(wid * NTT + (c - t * NTT) < VCH, s_t, 0.0)

        def grp(g, _):
            for e in range(16):
                row = rbase + g * 16 + e
                prods = []
                for h in range(4):
                    sa = plsc.bitcast(srcr[row, pl.ds(16 * h, 16)],
                                      jnp.bfloat16)
                    da = plsc.bitcast(dstr[row, pl.ds(16 * h, 16)],
                                      jnp.bfloat16)
                    prods.append(sa * da)
                accs = []
                for h in range(2):
                    u0, u1 = plsc.unpack(prods[2 * h] + prods[2 * h + 1],
                                         format=plsc.PackFormat.INTERLEAVED)
                    accs.append(u0 + u1)
                dots[pl.ds(e * 16, 16)] = accs[0] + accs[1]
            tot = plsc.load_gather(dots, [iota_sc])
            for j in range(1, 16):
                tot = tot + plsc.load_gather(
                    dots, [iota_sc + jnp.full((16,), j, jnp.int32)])
            wbase = c * CH + g * 16
            wall[pl.ds(wbase, 16)] = validf * tot
            for e in range(16):
                w16[rbase + g * 16 + e, :] = plsc.load_gather(
                    wall, [jnp.full((16,), wbase + e, jnp.int32)])
            return 0

        lax.fori_loop(0, CH // 16, grp, 0)
        fire(add_descs(c), add=True)
        return 0

    lax.fori_loop(0, NT, chunk_body, 0)
    drain(add_descs(NT - 1))

    for t in range(3):
        pltpu.sync_copy(wall.at[pl.ds(t * EPW, EPW)],
                        w_out.at[pl.ds(t * EPAD + wid * EPW, EPW)])

    plsc.subcore_barrier()
    pltpu.sync_copy(twacc.at[pl.ds(sid * RPT, RPT)],
                    tw_out.at[cid, pl.ds(sid * RPT, RPT)])


@functools.partial(
    pl.kernel,
    out_type=jax.ShapeDtypeStruct((2, NPAD, C), jnp.float32),
    mesh=_MESH,
    compiler_params=pltpu.CompilerParams(needs_layout_passes=False,
                                         use_tc_tiling_on_sc=False),
    scratch_types=[
        pltpu.VMEM((16, CH), jnp.int32),
        pltpu.VMEM((16, CH), jnp.int32),
        pltpu.VMEM((4 * 512,), jnp.float32),
        pltpu.VMEM((1024, C), jnp.float32),
        pltpu.VMEM_SHARED((NPAD, C), jnp.float32),
        pltpu.SemaphoreType.DMA((4,)),
        pltpu.SemaphoreType.DMA((2,)),
        pltpu.SemaphoreType.DMA,
    ],
)
def _layer_kernel(cur_hbm, src_hbm, dst_hbm, w_hbm, p_out,
                  srci, dsti, wv, rows, acc, semi, semg, sems):
    cid = lax.axis_index("c")
    sid = lax.axis_index("s")
    wid = sid * 2 + cid
    NT = 3 * EPW // 512

    def ebase(c):
        t = lax.div(c, NT // 3)
        return t * EPAD + wid * EPW + (c - t * (NT // 3)) * 512

    def idx_descs(c):
        base = ebase(c)
        slot = lax.rem(c, 4)
        ds = []
        for j in range(4):
            ds.append((src_hbm.at[pl.ds(base + j * CH, CH)],
                       srci.at[slot * 4 + j], semi.at[slot]))
            ds.append((dst_hbm.at[pl.ds(base + j * CH, CH)],
                       dsti.at[slot * 4 + j], semi.at[slot]))
        ds.append((w_hbm.at[pl.ds(base, 512)],
                   wv.at[pl.ds(slot * 512, 512)], semi.at[slot]))
        return ds

    def gat_descs(c):
        slot = lax.rem(c, 4)
        rbase = lax.rem(c, 2) * 512
        return [(cur_hbm.at[dsti.at[slot * 4 + j]],
                 rows.at[pl.ds(rbase + j * CH, CH)], semg.at[lax.rem(c, 2)])
                for j in range(4)]

    def sct_descs(c):
        slot = lax.rem(c, 4)
        rbase = lax.rem(c, 2) * 512
        return [(rows.at[pl.ds(rbase + j * CH, CH)],
                 acc.at[srci.at[slot * 4 + j]], sems)
                for j in range(4)]

    def fire(ds, add=False):
        for s, d, sem in ds:
            pltpu.async_copy(s, d, sem, add=add)

    def drain(ds):
        for s, d, sem in ds:
            pltpu.make_async_copy(s, d, sem).wait()

    zeros16 = jnp.zeros((16,), jnp.float32)

    fire(idx_descs(0))
    fire(idx_descs(1))

    def zrow(i, _):
        for k in range(C // 16):
            rows[i, pl.ds(16 * k, 16)] = zeros16
        return 0

    lax.fori_loop(0, CH, zrow, 0)

    def zcopy(j, _):
        pltpu.sync_copy(rows.at[pl.ds(0, CH)],
                        acc.at[pl.ds(sid * RPT + j * CH, CH)])
        return 0

    lax.fori_loop(0, RPT // CH, zcopy, 0)
    plsc.subcore_barrier()

    drain(idx_descs(0))
    fire(gat_descs(0))

    def chunk_body(c, _):

        @pl.when(c >= 1)
        def _():
            drain(sct_descs(c - 1))

        @pl.when(c + 2 < NT)
        def _():
            fire(idx_descs(c + 2))

        @pl.when(c + 1 < NT)
        def _():
            drain(idx_descs(c + 1))
            fire(gat_descs(c + 1))

        drain(gat_descs(c))
        rbase = lax.rem(c, 2) * 512
        wbase = lax.rem(c, 4) * 512

        def grp(g, _):
            for e in range(8):
                k = g * 8 + e
                wb = plsc.load_gather(
                    wv, [jnp.full((16,), wbase + k, jnp.int32)])
                row = rbase + k
                for q in range(C // 16):
                    rows[row, pl.ds(16 * q, 16)] = (
                        rows[row, pl.ds(16 * q, 16)] * wb)
            return 0

        lax.fori_loop(0, 512 // 8, grp, 0)
        fire(sct_descs(c), add=True)
        return 0

    lax.fori_loop(0, NT, chunk_body, 0)
    drain(sct_descs(NT - 1))

    plsc.subcore_barrier()
    pltpu.sync_copy(acc.at[pl.ds(sid * RPT, RPT)],
                    p_out.at[cid, pl.ds(sid * RPT, RPT)])


def _epilogue_body(p_ref, tw_ref, init_ref, out_ref):
    psum = p_ref[0] + p_ref[1]
    tw = tw_ref[0, :, 0:1] + tw_ref[1, :, 0:1]
    scale = jnp.where(tw > 0, 1.0 / tw, 1.0)
    out_ref[...] = ALPHA * psum * scale + (1.0 - ALPHA) * init_ref[...]


def _epilogue(p, twp, init_logits):
    return pl.pallas_call(
        _epilogue_body,
        grid=(N // BN,),
        in_specs=[
            pl.BlockSpec((2, BN, C), lambda i: (0, i, 0)),
            pl.BlockSpec((2, BN, 16), lambda i: (0, i, 0)),
            pl.BlockSpec((BN, C), lambda i: (i, 0)),
        ],
        out_specs=pl.BlockSpec((BN, C), lambda i: (i, 0)),
        out_shape=jax.ShapeDtypeStruct((N, C), jnp.float32),
    )(p, twp, init_logits)


def kernel(init_logits, features, src_connect, dst_connect, src_decorate,
           dst_decorate, src_next, dst_next, W, b, ln_gamma, ln_beta,
           ew_connect, ew_decorate, ew_next):
    pad = jnp.zeros((EPAD - E,), jnp.int32)
    src = jnp.concatenate([src_connect, pad, src_decorate, pad, src_next, pad])
    dst = jnp.concatenate([dst_connect, pad, dst_decorate, pad, dst_next, pad])
    ew = jnp.concatenate([ew_connect, ew_decorate, ew_next,
                          jnp.zeros((13,), jnp.float32)])

    that = _transform(features, W, b, ln_gamma, ln_beta)
    thatp = lax.bitcast_convert_type(
        that.astype(jnp.bfloat16).reshape(N, D // 2, 2), jnp.float32)
    w_e, twp3 = _weights_kernel(thatp, src.reshape(3840, CH),
                                dst.reshape(3840, CH), ew)

    cur = init_logits
    for _ in range(NUM_LAYERS):
        p = _layer_kernel(cur, src, dst, w_e)
        cur = _epilogue(p, twp3, init_logits)
    return cur


# speedup vs baseline: 3.8576x; 1.0388x over previous
import functools

import jax
import jax.numpy as jnp
from jax import lax
from jax.experimental import pallas as pl
from jax.experimental.pallas import tpu as pltpu
from jax.experimental.pallas import tpu_sc as plsc

N, D, C = 10000, 128, 64
NUM_LAYERS, ALPHA = 5, 0.5
NPAD = 10240
E = 160000
NW = 32
CH = 128
EPW = 5120
EPAD = EPW * NW
NCH = EPW // CH
VCH = E // CH
RPT = NPAD // 16
BN = 400


def _transform_body(f_ref, w_ref, b_ref, g_ref, be_ref, out_ref):
    x = f_ref[...]
    t = lax.dot_general(x, w_ref[...], (((1,), (1,)), ((), ())),
                        preferred_element_type=jnp.float32)
    t = t + b_ref[...]
    mu = jnp.mean(t, axis=-1, keepdims=True)
    var = jnp.mean((t - mu) ** 2, axis=-1, keepdims=True)
    t = (t - mu) * lax.rsqrt(var + 1e-5) * g_ref[...] + be_ref[...]
    t = jnp.maximum(t, 0.0)
    nrm = jnp.sqrt(jnp.sum(t * t, axis=-1, keepdims=True))
    out_ref[...] = t / jnp.maximum(nrm, 1e-8)


def _transform(features, W, b, ln_gamma, ln_beta):
    return pl.pallas_call(
        _transform_body,
        grid=(N // BN,),
        in_specs=[
            pl.BlockSpec((BN, D), lambda i: (i, 0)),
            pl.BlockSpec((D, D), lambda i: (0, 0)),
            pl.BlockSpec((1, D), lambda i: (0, 0)),
            pl.BlockSpec((1, D), lambda i: (0, 0)),
            pl.BlockSpec((1, D), lambda i: (0, 0)),
        ],
        out_specs=pl.BlockSpec((BN, D), lambda i: (i, 0)),
        out_shape=jax.ShapeDtypeStruct((N, D), jnp.float32),
    )(features, W, b.reshape(1, D), ln_gamma.reshape(1, D),
      ln_beta.reshape(1, D))


_MESH = plsc.VectorSubcoreMesh(core_axis_name="c", subcore_axis_name="s")


@functools.partial(
    pl.kernel,
    out_type=[
        jax.ShapeDtypeStruct((3 * EPAD,), jnp.float32),
        jax.ShapeDtypeStruct((2, NPAD, 16), jnp.float32),
    ],
    mesh=_MESH,
    compiler_params=pltpu.CompilerParams(needs_layout_passes=False,
                                         use_tc_tiling_on_sc=False),
    scratch_types=[
        pltpu.VMEM((120, CH), jnp.int32),
        pltpu.VMEM((120, CH), jnp.int32),
        pltpu.VMEM((512, D // 2), jnp.float32),
        pltpu.VMEM((512, D // 2), jnp.float32),
        pltpu.VMEM((256,), jnp.float32),
        pltpu.VMEM((120 * CH,), jnp.float32),
        pltpu.VMEM((256, 16), jnp.float32),
        pltpu.VMEM((16,), jnp.float32),
        pltpu.VMEM_SHARED((NPAD, 16), jnp.float32),
        pltpu.SemaphoreType.DMA,
        pltpu.SemaphoreType.DMA((4,)),
        pltpu.SemaphoreType.DMA((2,)),
    ],
)
def _weights_kernel(that_hbm, src2_hbm, dst2_hbm, ew_hbm, w_out, tw_out,
                    srci, dsti, srcr, dstr, dots, wall, w16, sigv, twacc,
                    semi, semg, sema):
    cid = lax.axis_index("c")
    sid = lax.axis_index("s")
    wid = sid * 2 + cid
    NT = 120
    NTT = NT // 3

    idescs = [(src2_hbm.at[pl.ds(t * 1280 + wid * NTT, NTT)],
               srci.at[pl.ds(t * NTT, NTT)], semi) for t in range(3)]
    idescs += [(dst2_hbm.at[pl.ds(t * 1280 + wid * NTT, NTT)],
                dsti.at[pl.ds(t * NTT, NTT)], semi) for t in range(3)]
    for s, d, sem in idescs:
        pltpu.async_copy(s, d, sem)

    pltpu.sync_copy(ew_hbm, sigv)
    sig = 1.0 / (1.0 + jnp.exp(-sigv[...]))
    s0, s1, s2 = sig[0], sig[1], sig[2]

    zeros16 = jnp.zeros((16,), jnp.float32)

    def zfill(i, _):
        w16[i, :] = zeros16
        return 0

    lax.fori_loop(0, CH, zfill, 0)

    def zcopy(j, _):
        pltpu.sync_copy(w16.at[pl.ds(0, CH)],
                        twacc.at[pl.ds(sid * RPT + j * CH, CH)])
        return 0

    lax.fori_loop(0, RPT // CH, zcopy, 0)
    plsc.subcore_barrier()

    for item in idescs:
        pltpu.make_async_copy(*item).wait()

    def gat_descs(c):
        rbase = lax.rem(c, 4) * CH
        return [(that_hbm.at[srci.at[c]],
                 srcr.at[pl.ds(rbase, CH)], semg.at[lax.rem(c, 4)]),
                (that_hbm.at[dsti.at[c]],
                 dstr.at[pl.ds(rbase, CH)], semg.at[lax.rem(c, 4)])]

    def add_descs(c):
        rbase = lax.rem(c, 2) * CH
        return [(w16.at[pl.ds(rbase, CH)],
                 twacc.at[srci.at[c]], sema.at[lax.rem(c, 2)])]

    def fire(ds, add=False):
        for s, d, sem in ds:
            pltpu.async_copy(s, d, sem, add=add)

    def drain(ds):
        for item in ds:
            pltpu.make_async_copy(*item).wait()

    fire(gat_descs(0))
    fire(gat_descs(1))
    fire(gat_descs(2))

    iota_sc = lax.iota(jnp.int32, 16) * 16

    def chunk_body(c, _):

        @pl.when(c + 3 < NT)
        def _():
            fire(gat_descs(c + 3))

        drain(gat_descs(c))

        @pl.when(c >= 1)
        def _():
            drain(add_descs(c - 1))

        rbase = lax.rem(c, 4) * CH
        t = lax.div(c, NTT)
        s_t = jnp.where(t == 0, s0, jnp.where(t == 1, s1, s2))
        validf = jnp.where(wid * NTT + (c - t * NTT) < VCH, s_t, 0.0)

        def grp(g, _):
            for e in range(16):
                row = rbase + g * 16 + e
                prods = []
                for h in range(4):
                    sa = plsc.bitcast(srcr[row, pl.ds(16 * h, 16)],
                                      jnp.bfloat16)
                    da = plsc.bitcast(dstr[row, pl.ds(16 * h, 16)],
                                      jnp.bfloat16)
                    prods.append(sa * da)
                accs = []
                for h in range(2):
                    u0, u1 = plsc.unpack(prods[2 * h] + prods[2 * h + 1],
                                         format=plsc.PackFormat.INTERLEAVED)
                    accs.append(u0 + u1)
                dots[pl.ds(e * 16, 16)] = accs[0] + accs[1]
            tot = plsc.load_gather(dots, [iota_sc])
            for j in range(1, 16):
                tot = tot + plsc.load_gather(
                    dots, [iota_sc + jnp.full((16,), j, jnp.int32)])
            wbase = c * CH + g * 16
            wall[pl.ds(wbase, 16)] = validf * tot
            rb2 = lax.rem(c, 2) * CH
            for e in range(16):
                w16[rb2 + g * 16 + e, :] = plsc.load_gather(
                    wall, [jnp.full((16,), wbase + e, jnp.int32)])
            return 0

        lax.fori_loop(0, CH // 16, grp, 0)
        fire(add_descs(c), add=True)
        return 0

    lax.fori_loop(0, NT, chunk_body, 0)
    drain(add_descs(NT - 1))

    for t in range(3):
        pltpu.sync_copy(wall.at[pl.ds(t * EPW, EPW)],
                        w_out.at[pl.ds(t * EPAD + wid * EPW, EPW)])

    plsc.subcore_barrier()
    pltpu.sync_copy(twacc.at[pl.ds(sid * RPT, RPT)],
                    tw_out.at[cid, pl.ds(sid * RPT, RPT)])


@functools.partial(
    pl.kernel,
    out_type=jax.ShapeDtypeStruct((2, NPAD, C), jnp.float32),
    mesh=_MESH,
    compiler_params=pltpu.CompilerParams(needs_layout_passes=False,
                                         use_tc_tiling_on_sc=False),
    scratch_types=[
        pltpu.VMEM((16, CH), jnp.int32),
        pltpu.VMEM((16, CH), jnp.int32),
        pltpu.VMEM((4 * 512,), jnp.float32),
        pltpu.VMEM((1024, C), jnp.float32),
        pltpu.VMEM_SHARED((NPAD, C), jnp.float32),
        pltpu.SemaphoreType.DMA((4,)),
        pltpu.SemaphoreType.DMA((2,)),
        pltpu.SemaphoreType.DMA,
    ],
)
def _layer_kernel(cur_hbm, src_hbm, dst_hbm, w_hbm, p_out,
                  srci, dsti, wv, rows, acc, semi, semg, sems):
    cid = lax.axis_index("c")
    sid = lax.axis_index("s")
    wid = sid * 2 + cid
    NT = 3 * EPW // 512

    def ebase(c):
        t = lax.div(c, NT // 3)
        return t * EPAD + wid * EPW + (c - t * (NT // 3)) * 512

    def idx_descs(c):
        base = ebase(c)
        slot = lax.rem(c, 4)
        ds = []
        for j in range(4):
            ds.append((src_hbm.at[pl.ds(base + j * CH, CH)],
                       srci.at[slot * 4 + j], semi.at[slot]))
            ds.append((dst_hbm.at[pl.ds(base + j * CH, CH)],
                       dsti.at[slot * 4 + j], semi.at[slot]))
        ds.append((w_hbm.at[pl.ds(base, 512)],
                   wv.at[pl.ds(slot * 512, 512)], semi.at[slot]))
        return ds

    def gat_descs(c):
        slot = lax.rem(c, 4)
        rbase = lax.rem(c, 2) * 512
        return [(cur_hbm.at[dsti.at[slot * 4 + j]],
                 rows.at[pl.ds(rbase + j * CH, CH)], semg.at[lax.rem(c, 2)])
                for j in range(4)]

    def sct_descs(c):
        slot = lax.rem(c, 4)
        rbase = lax.rem(c, 2) * 512
        return [(rows.at[pl.ds(rbase + j * CH, CH)],
                 acc.at[srci.at[slot * 4 + j]], sems)
                for j in range(4)]

    def fire(ds, add=False):
        for s, d, sem in ds:
            pltpu.async_copy(s, d, sem, add=add)

    def drain(ds):
        for s, d, sem in ds:
            pltpu.make_async_copy(s, d, sem).wait()

    zeros16 = jnp.zeros((16,), jnp.float32)

    fire(idx_descs(0))
    fire(idx_descs(1))

    def zrow(i, _):
        for k in range(C // 16):
            rows[i, pl.ds(16 * k, 16)] = zeros16
        return 0

    lax.fori_loop(0, CH, zrow, 0)

    def zcopy(j, _):
        pltpu.sync_copy(rows.at[pl.ds(0, CH)],
                        acc.at[pl.ds(sid * RPT + j * CH, CH)])
        return 0

    lax.fori_loop(0, RPT // CH, zcopy, 0)
    plsc.subcore_barrier()

    drain(idx_descs(0))
    fire(gat_descs(0))

    def chunk_body(c, _):

        @pl.when(c >= 1)
        def _():
            drain(sct_descs(c - 1))

        @pl.when(c + 2 < NT)
        def _():
            fire(idx_descs(c + 2))

        @pl.when(c + 1 < NT)
        def _():
            drain(idx_descs(c + 1))
            fire(gat_descs(c + 1))

        drain(gat_descs(c))
        rbase = lax.rem(c, 2) * 512
        wbase = lax.rem(c, 4) * 512

        def grp(g, _):
            for e in range(8):
                k = g * 8 + e
                wb = plsc.load_gather(
                    wv, [jnp.full((16,), wbase + k, jnp.int32)])
                row = rbase + k
                for q in range(C // 16):
                    rows[row, pl.ds(16 * q, 16)] = (
                        rows[row, pl.ds(16 * q, 16)] * wb)
            return 0

        lax.fori_loop(0, 512 // 8, grp, 0)
        fire(sct_descs(c), add=True)
        return 0

    lax.fori_loop(0, NT, chunk_body, 0)
    drain(sct_descs(NT - 1))

    plsc.subcore_barrier()
    pltpu.sync_copy(acc.at[pl.ds(sid * RPT, RPT)],
                    p_out.at[cid, pl.ds(sid * RPT, RPT)])


def _epilogue_body(p_ref, tw_ref, init_ref, out_ref):
    psum = p_ref[0] + p_ref[1]
    tw = tw_ref[0, :, 0:1] + tw_ref[1, :, 0:1]
    scale = jnp.where(tw > 0, 1.0 / tw, 1.0)
    out_ref[...] = ALPHA * psum * scale + (1.0 - ALPHA) * init_ref[...]


def _epilogue(p, twp, init_logits):
    return pl.pallas_call(
        _epilogue_body,
        grid=(N // BN,),
        in_specs=[
            pl.BlockSpec((2, BN, C), lambda i: (0, i, 0)),
            pl.BlockSpec((2, BN, 16), lambda i: (0, i, 0)),
            pl.BlockSpec((BN, C), lambda i: (i, 0)),
        ],
        out_specs=pl.BlockSpec((BN, C), lambda i: (i, 0)),
        out_shape=jax.ShapeDtypeStruct((N, C), jnp.float32),
    )(p, twp, init_logits)


def kernel(init_logits, features, src_connect, dst_connect, src_decorate,
           dst_decorate, src_next, dst_next, W, b, ln_gamma, ln_beta,
           ew_connect, ew_decorate, ew_next):
    pad = jnp.zeros((EPAD - E,), jnp.int32)
    src = jnp.concatenate([src_connect, pad, src_decorate, pad, src_next, pad])
    dst = jnp.concatenate([dst_connect, pad, dst_decorate, pad, dst_next, pad])
    ew = jnp.concatenate([ew_connect, ew_decorate, ew_next,
                          jnp.zeros((13,), jnp.float32)])

    that = _transform(features, W, b, ln_gamma, ln_beta)
    thatp = lax.bitcast_convert_type(
        that.astype(jnp.bfloat16).reshape(N, D // 2, 2), jnp.float32)
    w_e, twp3 = _weights_kernel(thatp, src.reshape(3840, CH),
                                dst.reshape(3840, CH), ew)

    cur = init_logits
    for _ in range(NUM_LAYERS):
        p = _layer_kernel(cur, src, dst, w_e)
        cur = _epilogue(p, twp3, init_logits)
    return cur
